# baseline (speedup 1.0000x reference)
import jax
import jax.numpy as jnp
from jax import lax
from jax.experimental import pallas as pl
from jax.experimental.pallas import tpu as pltpu
from jax.experimental.pallas import tpu_sc as plsc

_BATCH = 16384
_N_FIELDS = 26
_EMBED_DIM = 16
_R = _BATCH * _N_FIELDS
_NW = 32
_RW = _R // _NW
_SPB = 128
_NCHUNK = 13
_CROWS = _RW // _NCHUNK
_CSTREAMS = _CROWS // _SPB

_C0 = 1.1447097560713972e-05
_C1 = 0.9991664010110775
_C2 = -0.48969909032091086
_C3 = 0.28382318306553606
_C4 = -0.1299571976585037
_C5 = 0.0298087652435521


def _softplus16(v):
    e = jnp.exp(-jnp.abs(v))
    p = jnp.float32(_C5)
    for c in (_C4, _C3, _C2, _C1, _C0):
        p = p * e + jnp.float32(c)
    return jnp.maximum(v, jnp.float32(0.0)) + p


def _body(x_hbm, params_hbm, out_hbm, idx_v, rows_v, sem):
    wid = lax.axis_index("s") * 2 + lax.axis_index("c")
    for g in range(_NCHUNK):
        srow = wid * (_RW // _SPB) + g * _CSTREAMS
        pltpu.sync_copy(x_hbm.at[pl.ds(srow, _CSTREAMS)], idx_v)
        copies = [
            pltpu.make_async_copy(
                params_hbm.at[idx_v.at[j]],
                rows_v.at[pl.ds(j * _SPB, _SPB)],
                sem,
            )
            for j in range(_CSTREAMS)
        ]
        for c in copies:
            c.start()
        for c in copies:
            c.wait()

        def compute(i, carry):
            rows_v[i, :] = _softplus16(rows_v[i, :])
            return carry

        lax.fori_loop(0, _CROWS, compute, 0)
        pltpu.sync_copy(
            rows_v, out_hbm.at[pl.ds(wid * _RW + g * _CROWS, _CROWS)]
        )


_mesh = plsc.VectorSubcoreMesh(core_axis_name="c", subcore_axis_name="s")

_gather_softplus = pl.kernel(
    _body,
    out_type=jax.ShapeDtypeStruct((_R, _EMBED_DIM), jnp.float32),
    mesh=_mesh,
    scratch_types=[
        pltpu.VMEM((_CSTREAMS, _SPB), jnp.int32),
        pltpu.VMEM((_CROWS, _EMBED_DIM), jnp.float32),
        pltpu.SemaphoreType.DMA,
    ],
    compiler_params=pltpu.CompilerParams(use_tc_tiling_on_sc=False),
)


def kernel(x, params):
    x32 = x.astype(jnp.int32).reshape(_R // _SPB, _SPB)
    out = _gather_softplus(x32, params)
    return out.reshape(_BATCH, _N_FIELDS, _EMBED_DIM)


# speedup vs baseline: 1.2762x; 1.2762x over previous
import jax
import jax.numpy as jnp
from jax import lax
from jax.experimental import pallas as pl
from jax.experimental.pallas import tpu as pltpu
from jax.experimental.pallas import tpu_sc as plsc

_BATCH = 16384
_N_FIELDS = 26
_EMBED_DIM = 16
_R = _BATCH * _N_FIELDS
_NW = 32
_RW = _R // _NW
_SPB = 128
_NCHUNK = 13
_CROWS = _RW // _NCHUNK
_CSTREAMS = _CROWS // _SPB
_UNROLL = 4

_C0 = 1.1447097560713972e-05
_C1 = 0.9991664010110775
_C2 = -0.48969909032091086
_C3 = 0.28382318306553606
_C4 = -0.1299571976585037
_C5 = 0.0298087652435521


def _softplus16(v):
    e = jnp.exp(-jnp.abs(v))
    p = jnp.float32(_C5)
    for c in (_C4, _C3, _C2, _C1, _C0):
        p = p * e + jnp.float32(c)
    return jnp.maximum(v, jnp.float32(0.0)) + p


def _body(x_hbm, params_hbm, out_hbm, idx_v, rows_v, gsem, osem):
    wid = lax.axis_index("s") * 2 + lax.axis_index("c")
    base_stream = wid * (_RW // _SPB)

    pltpu.sync_copy(x_hbm.at[pl.ds(base_stream, _NCHUNK * _CSTREAMS)], idx_v)

    def gathers(g, buf):
        return [
            pltpu.make_async_copy(
                params_hbm.at[idx_v.at[g * _CSTREAMS + j]],
                rows_v.at[buf].at[pl.ds(j * _SPB, _SPB)],
                gsem.at[buf],
            )
            for j in range(_CSTREAMS)
        ]

    def out_copy(g, buf):
        return pltpu.make_async_copy(
            rows_v.at[buf],
            out_hbm.at[pl.ds(wid * _RW + g * _CROWS, _CROWS)],
            osem.at[buf],
        )

    for c in gathers(0, 0):
        c.start()

    for g in range(_NCHUNK):
        buf = g % 2
        nbuf = 1 - buf
        if g + 1 < _NCHUNK:
            if g >= 1:
                out_copy(g - 1, nbuf).wait()
            for c in gathers(g + 1, nbuf):
                c.start()
        for c in gathers(g, buf):
            c.wait()

        def compute(i, carry):
            for u in range(_UNROLL):
                r = i * _UNROLL + u
                rows_v[buf, r, :] = _softplus16(rows_v[buf, r, :])
            return carry

        lax.fori_loop(0, _CROWS // _UNROLL, compute, 0)
        out_copy(g, buf).start()

    out_copy(_NCHUNK - 2, (_NCHUNK - 2) % 2).wait()
    out_copy(_NCHUNK - 1, (_NCHUNK - 1) % 2).wait()


_mesh = plsc.VectorSubcoreMesh(core_axis_name="c", subcore_axis_name="s")

_gather_softplus = pl.kernel(
    _body,
    out_type=jax.ShapeDtypeStruct((_R, _EMBED_DIM), jnp.float32),
    mesh=_mesh,
    scratch_types=[
        pltpu.VMEM((_NCHUNK * _CSTREAMS, _SPB), jnp.int32),
        pltpu.VMEM((2, _CROWS, _EMBED_DIM), jnp.float32),
        pltpu.SemaphoreType.DMA((2,)),
        pltpu.SemaphoreType.DMA((2,)),
    ],
    compiler_params=pltpu.CompilerParams(use_tc_tiling_on_sc=False),
)


def kernel(x, params):
    x32 = x.astype(jnp.int32).reshape(_R // _SPB, _SPB)
    out = _gather_softplus(x32, params)
    return out.reshape(_BATCH, _N_FIELDS, _EMBED_DIM)


# speedup vs baseline: 1.5155x; 1.1874x over previous
import jax
import jax.numpy as jnp
from jax import lax
from jax.experimental import pallas as pl
from jax.experimental.pallas import tpu as pltpu
from jax.experimental.pallas import tpu_sc as plsc

_BATCH = 16384
_N_FIELDS = 26
_EMBED_DIM = 16
_R = _BATCH * _N_FIELDS
_NW = 32
_SPB = 128
_NBLOCKS = _R // _SPB
_BW = _NBLOCKS // _NW
_CBLK = 8
_NCHUNK = _BW // _CBLK
_CROWS = _CBLK * _SPB
_NBB = _BATCH // _SPB

_C0 = 1.1447097560713972e-05
_C1 = 0.9991664010110775
_C2 = -0.48969909032091086
_C3 = 0.28382318306553606
_C4 = -0.1299571976585037
_C5 = 0.0298087652435521


def _softplus16(v):
    e = jnp.exp(-jnp.abs(v))
    p = jnp.float32(_C5)
    for c in (_C4, _C3, _C2, _C1, _C0):
        p = p * e + jnp.float32(c)
    return jnp.maximum(v, jnp.float32(0.0)) + p


def _body(x_hbm, params_hbm, out_hbm, idx_v, g_v, t_v, gsem, osem):
    wid = lax.axis_index("s") * 2 + lax.axis_index("c")
    t0 = wid * _BW

    pltpu.sync_copy(x_hbm.at[pl.ds(t0, _BW)], idx_v)

    lanes = lax.iota(jnp.int32, 16)

    def gathers(g, buf):
        return [
            pltpu.make_async_copy(
                params_hbm.at[idx_v.at[g * _CBLK + j]],
                g_v.at[buf].at[pl.ds(j * _SPB, _SPB)],
                gsem.at[buf],
            )
            for j in range(_CBLK)
        ]

    def out_copies(g, buf):
        tbase = t0 + g * _CBLK
        f = tbase // _NBB
        c0 = tbase % _NBB
        return [
            pltpu.make_async_copy(
                t_v.at[buf].at[pl.ds(r * (_CBLK * 1024), _CBLK * 1024)],
                out_hbm.at[pl.ds((f * 256 + r * 128 + c0) * 1024, _CBLK * 1024)],
                osem.at[buf],
            )
            for r in range(2)
        ]

    for c in gathers(0, 0):
        c.start()

    for g in range(_NCHUNK):
        buf = g % 2
        if g + 1 < _NCHUNK:
            for c in gathers(g + 1, 1 - buf):
                c.start()
        for c in gathers(g, buf):
            c.wait()
        if g >= 2:
            for c in out_copies(g - 2, buf):
                c.wait()

        def compute(j, carry):
            r = j >> 6
            d = ((j >> 3) & 7) | (r << 3)
            l0 = (j & 7) << 4
            row = l0 + lanes
            dsel = jnp.full((16,), d, jnp.int32)
            for b in range(_CBLK):
                v = plsc.load_gather(
                    g_v.at[buf], [row + (b * _SPB), dsel]
                )
                t_v[buf, pl.ds(b * 1024 + (j << 4) + r * 7168, 16)] = (
                    _softplus16(v)
                )
            return carry

        lax.fori_loop(0, 128, compute, 0)
        for c in out_copies(g, buf):
            c.start()

    for c in out_copies(_NCHUNK - 2, (_NCHUNK - 2) % 2):
        c.wait()
    for c in out_copies(_NCHUNK - 1, (_NCHUNK - 1) % 2):
        c.wait()


_mesh = plsc.VectorSubcoreMesh(core_axis_name="c", subcore_axis_name="s")

_gather_softplus = pl.kernel(
    _body,
    out_type=jax.ShapeDtypeStruct((_R * _EMBED_DIM,), jnp.float32),
    mesh=_mesh,
    scratch_types=[
        pltpu.VMEM((_BW, _SPB), jnp.int32),
        pltpu.VMEM((2, _CROWS, _EMBED_DIM), jnp.float32),
        pltpu.VMEM((2, 2 * _CBLK * 1024), jnp.float32),
        pltpu.SemaphoreType.DMA((2,)),
        pltpu.SemaphoreType.DMA((2,)),
    ],
    compiler_params=pltpu.CompilerParams(
        use_tc_tiling_on_sc=False, needs_layout_passes=False
    ),
)


def kernel(x, params):
    xt = x.astype(jnp.int32).T.reshape(_NBLOCKS, _SPB)
    out = _gather_softplus(xt, params)
    return (
        out.reshape(_N_FIELDS, 2, _NBB, 8, _SPB)
        .transpose(2, 4, 0, 1, 3)
        .reshape(_BATCH, _N_FIELDS, _EMBED_DIM)
    )
